# baseline (speedup 1.0000x reference)
import functools

import jax
import jax.numpy as jnp
from jax import lax
from jax.experimental import pallas as pl
from jax.experimental.pallas import tpu as pltpu

_TN = 1792


def _vq_block(x_ref, w_ref, fio_ref, d_ref, e_ref, q_ref, l_ref, *, hw, wdt, k):
    g = pl.program_id(0)
    x = x_ref[...]
    w = w_ref[...]
    fio = fio_ref[...]
    s = x * x + w * w
    d = s - x * (w + w)
    d_ref[...] = d
    mind = jnp.min(d, axis=1, keepdims=True)
    idxf = jnp.min(jnp.where(d == mind, fio, jnp.float32(k)), axis=1,
                   keepdims=True)
    idx = idxf.astype(jnp.int32)
    e_ref[...] = idx

    @pl.when((g % (hw // _TN)) == 0)
    def _():
        p = lax.broadcasted_iota(jnp.int32, (_TN, 1), 0)
        h = p // wdt
        lim = jnp.where(h < 4, 2 * h + 1, k - 1)
        e_ref[...] = jnp.minimum(idx, lim)

    ef = e_ref[...].astype(jnp.float32)
    wv = jnp.sum(jnp.where(fio == ef, w, 0.0), axis=1, keepdims=True)
    q_ref[...] = x + (wv - x)
    l_ref[...] = jnp.sum((wv - x) ** 2).reshape(1, 1, 1)


def kernel(input, weight):
    b, c, hgt, wdt = input.shape
    k = weight.shape[0]
    n = b * c * hgt * wdt
    x = input.reshape(n, 1)
    wt = weight.reshape(1, k)
    fio = jnp.arange(k, dtype=jnp.float32).reshape(1, k)
    grid = n // _TN
    d, e, q, l = pl.pallas_call(
        functools.partial(_vq_block, hw=hgt * wdt, wdt=wdt, k=k),
        grid=(grid,),
        in_specs=[
            pl.BlockSpec((_TN, 1), lambda g: (g, 0)),
            pl.BlockSpec((1, k), lambda g: (0, 0)),
            pl.BlockSpec((1, k), lambda g: (0, 0)),
        ],
        out_specs=[
            pl.BlockSpec((_TN, k), lambda g: (g, 0)),
            pl.BlockSpec((_TN, 1), lambda g: (g, 0)),
            pl.BlockSpec((_TN, 1), lambda g: (g, 0)),
            pl.BlockSpec((1, 1, 1), lambda g: (g, 0, 0)),
        ],
        out_shape=[
            jax.ShapeDtypeStruct((n, k), jnp.float32),
            jax.ShapeDtypeStruct((n, 1), jnp.int32),
            jax.ShapeDtypeStruct((n, 1), jnp.float32),
            jax.ShapeDtypeStruct((grid, 1, 1), jnp.float32),
        ],
        compiler_params=pltpu.CompilerParams(
            dimension_semantics=("parallel",)),
    )(x, wt, fio)
    encoding = e.reshape(b, hgt, wdt)
    quantized_ste = q.reshape(b, c, hgt, wdt)
    loss = jnp.sum(l) * (2.0 / n)
    return quantized_ste, encoding, d, loss


# speedup vs baseline: 1.0472x; 1.0472x over previous
import functools

import jax
import jax.numpy as jnp
from jax import lax
from jax.experimental import pallas as pl
from jax.experimental.pallas import tpu as pltpu

_TN = 3584


def _vq_block(x_ref, w_ref, fio_ref, d_ref, e_ref, q_ref, l_ref, *, hw, wdt, k):
    g = pl.program_id(0)
    x = x_ref[...]
    w = w_ref[...]
    fio = fio_ref[...]
    s = x * x + w * w
    d = s - x * (w + w)
    d_ref[...] = d
    mind = jnp.min(d, axis=1, keepdims=True)
    idxf = jnp.min(jnp.where(d == mind, fio, jnp.float32(k)), axis=1,
                   keepdims=True)
    idx = idxf.astype(jnp.int32)
    e_ref[...] = idx

    @pl.when((g % (hw // _TN)) == 0)
    def _():
        p = lax.broadcasted_iota(jnp.int32, (_TN, 1), 0)
        h = p // wdt
        lim = jnp.where(h < 4, 2 * h + 1, k - 1)
        e_ref[...] = jnp.minimum(idx, lim)

    ef = e_ref[...].astype(jnp.float32)
    wv = jnp.sum(jnp.where(fio == ef, w, 0.0), axis=1, keepdims=True)
    q_ref[...] = x + (wv - x)
    l_ref[...] = jnp.sum((wv - x) ** 2).reshape(1, 1, 1)


def kernel(input, weight):
    b, c, hgt, wdt = input.shape
    k = weight.shape[0]
    n = b * c * hgt * wdt
    x = input.reshape(n, 1)
    wt = weight.reshape(1, k)
    fio = jnp.arange(k, dtype=jnp.float32).reshape(1, k)
    grid = n // _TN
    d, e, q, l = pl.pallas_call(
        functools.partial(_vq_block, hw=hgt * wdt, wdt=wdt, k=k),
        grid=(grid,),
        in_specs=[
            pl.BlockSpec((_TN, 1), lambda g: (g, 0)),
            pl.BlockSpec((1, k), lambda g: (0, 0)),
            pl.BlockSpec((1, k), lambda g: (0, 0)),
        ],
        out_specs=[
            pl.BlockSpec((_TN, k), lambda g: (g, 0)),
            pl.BlockSpec((_TN, 1), lambda g: (g, 0)),
            pl.BlockSpec((_TN, 1), lambda g: (g, 0)),
            pl.BlockSpec((1, 1, 1), lambda g: (g, 0, 0)),
        ],
        out_shape=[
            jax.ShapeDtypeStruct((n, k), jnp.float32),
            jax.ShapeDtypeStruct((n, 1), jnp.int32),
            jax.ShapeDtypeStruct((n, 1), jnp.float32),
            jax.ShapeDtypeStruct((grid, 1, 1), jnp.float32),
        ],
        compiler_params=pltpu.CompilerParams(
            dimension_semantics=("arbitrary",)),
    )(x, wt, fio)
    encoding = e.reshape(b, hgt, wdt)
    quantized_ste = q.reshape(b, c, hgt, wdt)
    loss = jnp.sum(l) * (2.0 / n)
    return quantized_ste, encoding, d, loss


# speedup vs baseline: 1.0492x; 1.0020x over previous
import functools

import jax
import jax.numpy as jnp
from jax import lax
from jax.experimental import pallas as pl
from jax.experimental.pallas import tpu as pltpu

_TN = 7168


def _vq_block(x_ref, w_ref, fio_ref, d_ref, e_ref, q_ref, l_ref, *, hw, wdt, k):
    g = pl.program_id(0)
    x = x_ref[...]
    w = w_ref[...]
    fio = fio_ref[...]
    s = x * x + w * w
    d = s - x * (w + w)
    d_ref[...] = d
    mind = jnp.min(d, axis=1, keepdims=True)
    idxf = jnp.min(jnp.where(d == mind, fio, jnp.float32(k)), axis=1,
                   keepdims=True)
    idx = idxf.astype(jnp.int32)
    e_ref[...] = idx

    @pl.when((g % (hw // _TN)) == 0)
    def _():
        p = lax.broadcasted_iota(jnp.int32, (_TN, 1), 0)
        h = p // wdt
        lim = jnp.where(h < 4, 2 * h + 1, k - 1)
        e_ref[...] = jnp.minimum(idx, lim)

    ef = e_ref[...].astype(jnp.float32)
    wv = jnp.sum(jnp.where(fio == ef, w, 0.0), axis=1, keepdims=True)
    q_ref[...] = x + (wv - x)
    l_ref[...] = jnp.sum((wv - x) ** 2).reshape(1, 1, 1)


def kernel(input, weight):
    b, c, hgt, wdt = input.shape
    k = weight.shape[0]
    n = b * c * hgt * wdt
    x = input.reshape(n, 1)
    wt = weight.reshape(1, k)
    fio = jnp.arange(k, dtype=jnp.float32).reshape(1, k)
    grid = n // _TN
    d, e, q, l = pl.pallas_call(
        functools.partial(_vq_block, hw=hgt * wdt, wdt=wdt, k=k),
        grid=(grid,),
        in_specs=[
            pl.BlockSpec((_TN, 1), lambda g: (g, 0)),
            pl.BlockSpec((1, k), lambda g: (0, 0)),
            pl.BlockSpec((1, k), lambda g: (0, 0)),
        ],
        out_specs=[
            pl.BlockSpec((_TN, k), lambda g: (g, 0)),
            pl.BlockSpec((_TN, 1), lambda g: (g, 0)),
            pl.BlockSpec((_TN, 1), lambda g: (g, 0)),
            pl.BlockSpec((1, 1, 1), lambda g: (g, 0, 0)),
        ],
        out_shape=[
            jax.ShapeDtypeStruct((n, k), jnp.float32),
            jax.ShapeDtypeStruct((n, 1), jnp.int32),
            jax.ShapeDtypeStruct((n, 1), jnp.float32),
            jax.ShapeDtypeStruct((grid, 1, 1), jnp.float32),
        ],
        compiler_params=pltpu.CompilerParams(
            dimension_semantics=("arbitrary",),
            vmem_limit_bytes=128 * 1024 * 1024),
    )(x, wt, fio)
    encoding = e.reshape(b, hgt, wdt)
    quantized_ste = q.reshape(b, c, hgt, wdt)
    loss = jnp.sum(l) * (2.0 / n)
    return quantized_ste, encoding, d, loss
